# baseline (speedup 1.0000x reference)
import functools

import jax
import jax.numpy as jnp
from jax import lax
from jax.experimental import pallas as pl
from jax.experimental.pallas import tpu as pltpu
from jax.experimental.pallas import tpu_sc as plsc

N = 10000
E = 320000
D1 = 16
D2P = 48
NC = 40
CHUNK = 80
ROWS = E // CHUNK
RPT = ROWS // 32
NSL = 640
NSL_LAST = N - 15 * NSL

_mesh = plsc.VectorSubcoreMesh(core_axis_name="c", subcore_axis_name="s")


def _acc_slices(s, copy_640, copy_400):
    @pl.when(s < 15)
    def _():
        copy_640(s * NSL)

    @pl.when(s == 15)
    def _():
        copy_400(15 * NSL)


def _deg_body(dst_hbm, aux_hbm, out_hbm, idx_v, ones_v, acc_a, acc_b, sem):
    accs = (acc_a, acc_b)
    c = lax.axis_index("c")
    s = lax.axis_index("s")
    w = c * 16 + s
    pltpu.sync_copy(dst_hbm.at[w], idx_v)
    pltpu.sync_copy(aux_hbm.at[pl.ds(N, CHUNK)], ones_v)
    for acc in accs:
        _acc_slices(
            s,
            lambda o, a=acc: pltpu.sync_copy(aux_hbm.at[pl.ds(o, NSL)],
                                             a.at[pl.ds(o, NSL)]),
            lambda o, a=acc: pltpu.sync_copy(aux_hbm.at[pl.ds(o, NSL_LAST)],
                                             a.at[pl.ds(o, NSL_LAST)]),
        )
    plsc.subcore_barrier()

    def body(i, carry):
        pltpu.async_copy(ones_v, acc_a.at[idx_v.at[2 * i]], sem.at[0],
                         add=True)
        pltpu.async_copy(ones_v, acc_b.at[idx_v.at[2 * i + 1]], sem.at[1],
                         add=True)
        pltpu.make_async_copy(ones_v, acc_a.at[idx_v.at[0]], sem.at[0]).wait()
        pltpu.make_async_copy(ones_v, acc_b.at[idx_v.at[0]], sem.at[1]).wait()
        return carry

    lax.fori_loop(0, RPT // 2, body, 0)
    pltpu.sync_copy(ones_v, acc_a.at[idx_v.at[RPT - 1]], add=True)
    plsc.subcore_barrier()
    for j, acc in enumerate(accs):
        _acc_slices(
            s,
            lambda o, a=acc, j=j: pltpu.sync_copy(
                a.at[pl.ds(o, NSL)],
                out_hbm.at[pl.ds((2 * c + j) * N + o, NSL)]),
            lambda o, a=acc, j=j: pltpu.sync_copy(
                a.at[pl.ds(o, NSL_LAST)],
                out_hbm.at[pl.ds((2 * c + j) * N + o, NSL_LAST)]),
        )


_SC_PARAMS = pltpu.CompilerParams(use_tc_tiling_on_sc=False)

_deg_call = pl.kernel(
    _deg_body,
    out_type=jax.ShapeDtypeStruct((4 * N, 1), jnp.float32),
    mesh=_mesh,
    compiler_params=_SC_PARAMS,
    scratch_types=[
        pltpu.VMEM((RPT, CHUNK), jnp.int32),
        pltpu.VMEM((CHUNK, 1), jnp.float32),
        pltpu.VMEM_SHARED((N, 1), jnp.float32),
        pltpu.VMEM_SHARED((N, 1), jnp.float32),
        pltpu.SemaphoreType.DMA((2,)),
    ],
)


PF = 5
NBUF = 10
NGRP = (RPT - 2 * PF) // NBUF


def _scat_body(src_hbm, dst_hbm, g_hbm, out_hbm, idx_s, idx_d, buf, acc_a,
               acc_b, gsem, ssem):
    bufs = [buf.at[pl.ds(b * CHUNK, CHUNK)] for b in range(NBUF)]
    gsems = [gsem.at[b] for b in range(NBUF)]
    accs = (acc_a, acc_b)
    c = lax.axis_index("c")
    s = lax.axis_index("s")
    w = c * 16 + s
    pltpu.sync_copy(src_hbm.at[w], idx_s)
    pltpu.sync_copy(dst_hbm.at[w], idx_d)
    for acc in accs:
        _acc_slices(
            s,
            lambda o, a=acc: pltpu.sync_copy(g_hbm.at[pl.ds(o, NSL)],
                                             a.at[pl.ds(o, NSL)]),
            lambda o, a=acc: pltpu.sync_copy(g_hbm.at[pl.ds(o, NSL_LAST)],
                                             a.at[pl.ds(o, NSL_LAST)]),
        )
    plsc.subcore_barrier()

    def fire_gather(wi, b):
        pltpu.async_copy(g_hbm.at[idx_s.at[wi]], bufs[b], gsems[b])

    def wait_gather(wi, b):
        pltpu.make_async_copy(g_hbm.at[idx_s.at[wi]], bufs[b],
                              gsems[b]).wait()

    def fire_scatter(wi, b, p):
        pltpu.async_copy(bufs[b], accs[p].at[idx_d.at[wi]], ssem.at[p],
                         add=True)

    def wait_scatter(p):
        pltpu.make_async_copy(bufs[0], accs[p].at[idx_d.at[0]],
                              ssem.at[p]).wait()

    def substep(wi, b, bp, p, swait):
        wait_gather(wi, b)
        if swait:
            wait_scatter(p)
        fire_scatter(wi, b, p)
        if bp is not None:
            fire_gather(wi + PF, bp)

    for wi in range(PF):
        fire_gather(wi, wi)
    for wi in range(PF):
        substep(wi, wi, wi + PF, wi % 2, wi >= 2)

    def body(g, carry):
        w0 = g * NBUF + PF
        for k in range(NBUF):
            substep(w0 + k, (PF + k) % NBUF, k % NBUF, (PF + k) % 2, True)
        return carry

    lax.fori_loop(0, NGRP, body, 0)
    base = NGRP * NBUF + PF
    for k in range(PF):
        wi = base + k
        substep(wi, wi % NBUF, (wi + PF) % NBUF, wi % 2, True)
    for k in range(PF):
        wi = base + PF + k
        substep(wi, wi % NBUF, None, wi % 2, True)
    for p in range(2):
        wait_scatter(p)
    plsc.subcore_barrier()
    for j, acc in enumerate(accs):
        _acc_slices(
            s,
            lambda o, a=acc, j=j: pltpu.sync_copy(
                a.at[pl.ds(o, NSL)],
                out_hbm.at[pl.ds((2 * c + j) * N + o, NSL)]),
            lambda o, a=acc, j=j: pltpu.sync_copy(
                a.at[pl.ds(o, NSL_LAST)],
                out_hbm.at[pl.ds((2 * c + j) * N + o, NSL_LAST)]),
        )


def _make_scat(d):
    return pl.kernel(
        _scat_body,
        out_type=jax.ShapeDtypeStruct((4 * N, d), jnp.float32),
        mesh=_mesh,
        compiler_params=_SC_PARAMS,
        scratch_types=[
            pltpu.VMEM((RPT, CHUNK), jnp.int32),
            pltpu.VMEM((RPT, CHUNK), jnp.int32),
            pltpu.VMEM((NBUF * CHUNK, d), jnp.float32),
            pltpu.VMEM_SHARED((N, d), jnp.float32),
            pltpu.VMEM_SHARED((N, d), jnp.float32),
            pltpu.SemaphoreType.DMA((NBUF,)),
            pltpu.SemaphoreType.DMA((2,)),
        ],
    )


_scat16 = _make_scat(D1)
_scat48 = _make_scat(D2P)


def _tc1_body(deg_ref, x_ref, w1_ref, g1_ref, dinv_ref):
    degp = deg_ref[...]
    deg = (degp[:N, :] + degp[N:2 * N, :] + degp[2 * N:3 * N, :]
           + degp[3 * N:, :] + 1.0)
    dinv = lax.rsqrt(deg)
    h = jnp.dot(x_ref[...], w1_ref[...], preferred_element_type=jnp.float32)
    g1_ref[...] = h * dinv
    dinv_ref[...] = dinv


_tc1 = pl.pallas_call(
    _tc1_body,
    out_shape=(
        jax.ShapeDtypeStruct((N, D1), jnp.float32),
        jax.ShapeDtypeStruct((N, 1), jnp.float32),
    ),
)


def _tc2_body(acc_ref, g1_ref, dinv_ref, w2_ref, b1_ref, g2_ref):
    acc = acc_ref[...]
    g1 = g1_ref[...]
    dinv = dinv_ref[...]
    agg = (acc[:N, :] + acc[N:2 * N, :] + acc[2 * N:3 * N, :]
           + acc[3 * N:, :] - 3.0 * g1)
    z = jnp.maximum(agg * dinv + b1_ref[...], 0.0)
    h2 = jnp.dot(z, w2_ref[...], preferred_element_type=jnp.float32)
    g2_ref[:, :NC] = h2 * dinv
    g2_ref[:, NC:] = jnp.zeros((N, D2P - NC), jnp.float32)


_tc2 = pl.pallas_call(
    _tc2_body,
    out_shape=jax.ShapeDtypeStruct((N, D2P), jnp.float32),
)


def _tc3_body(acc_ref, g2_ref, dinv_ref, b2_ref, out_ref):
    acc = acc_ref[...]
    agg = (acc[:N, :] + acc[N:2 * N, :] + acc[2 * N:3 * N, :]
           + acc[3 * N:, :] - 3.0 * g2_ref[...])
    o = agg[:, :NC] * dinv_ref[...] + b2_ref[...]
    m = jnp.max(o, axis=1, keepdims=True)
    e = o - m
    lse = jnp.log(jnp.sum(jnp.exp(e), axis=1, keepdims=True))
    out_ref[...] = e - lse


_tc3 = pl.pallas_call(
    _tc3_body,
    out_shape=jax.ShapeDtypeStruct((N, NC), jnp.float32),
)


def kernel(x, edge_index, W1, b1, W2, b2):
    src2d = edge_index[0].reshape(32, RPT, CHUNK)
    dst2d = edge_index[1].reshape(32, RPT, CHUNK)
    aux = jnp.concatenate(
        [jnp.zeros((N, 1), jnp.float32), jnp.ones((CHUNK, 1), jnp.float32)])
    degp = _deg_call(dst2d, aux)
    g1, dinv = _tc1(degp, x, W1)
    acc1 = _scat16(src2d, dst2d, g1)
    g2 = _tc2(acc1, g1, dinv, W2, b1.reshape(1, D1))
    acc2 = _scat48(src2d, dst2d, g2)
    return _tc3(acc2, g2, dinv, b2.reshape(1, NC))


# speedup vs baseline: 1.1059x; 1.1059x over previous
import jax
import jax.numpy as jnp
from jax import lax
from jax.experimental import pallas as pl
from jax.experimental.pallas import tpu as pltpu
from jax.experimental.pallas import tpu_sc as plsc

N = 10000
E = 320000
D1 = 16
D2P = 48
NC = 40
NPAD = 16
NP = N + NPAD
CHUNK = 128
RPT = 79
EP = 32 * RPT * CHUNK
NSL = 640
NSL_LAST = NP - 15 * NSL

PF = 5
NBUF = 10
NGRP = (RPT - 2 * PF) // NBUF
TAIL = RPT - PF - NGRP * NBUF

_mesh = plsc.VectorSubcoreMesh(core_axis_name="c", subcore_axis_name="s")
_SC_PARAMS = pltpu.CompilerParams(use_tc_tiling_on_sc=False)


def _acc_slices(s, copy_main, copy_last):
    @pl.when(s < 15)
    def _():
        copy_main(s * NSL)

    @pl.when(s == 15)
    def _():
        copy_last(15 * NSL)


def _deg_body(dst_hbm, aux_hbm, out_hbm, idx_v, ones_v, acc_sh, sem):
    c = lax.axis_index("c")
    s = lax.axis_index("s")
    w = c * 16 + s
    pltpu.sync_copy(dst_hbm.at[w], idx_v)
    pltpu.sync_copy(aux_hbm.at[pl.ds(NP, CHUNK)], ones_v)
    _acc_slices(
        s,
        lambda o: pltpu.sync_copy(aux_hbm.at[pl.ds(o, NSL)],
                                  acc_sh.at[pl.ds(o, NSL)]),
        lambda o: pltpu.sync_copy(aux_hbm.at[pl.ds(o, NSL_LAST)],
                                  acc_sh.at[pl.ds(o, NSL_LAST)]),
    )
    plsc.subcore_barrier()

    def body(i, carry):
        pltpu.sync_copy(ones_v, acc_sh.at[idx_v.at[i]], add=True)
        return carry

    lax.fori_loop(0, RPT, body, 0)
    plsc.subcore_barrier()
    _acc_slices(
        s,
        lambda o: pltpu.sync_copy(acc_sh.at[pl.ds(o, NSL)],
                                  out_hbm.at[pl.ds(c * NP + o, NSL)]),
        lambda o: pltpu.sync_copy(acc_sh.at[pl.ds(o, NSL_LAST)],
                                  out_hbm.at[pl.ds(c * NP + o, NSL_LAST)]),
    )


_deg_call = pl.kernel(
    _deg_body,
    out_type=jax.ShapeDtypeStruct((2 * NP, 1), jnp.float32),
    mesh=_mesh,
    compiler_params=_SC_PARAMS,
    scratch_types=[
        pltpu.VMEM((RPT, CHUNK), jnp.int32),
        pltpu.VMEM((CHUNK, 1), jnp.float32),
        pltpu.VMEM_SHARED((NP, 1), jnp.float32),
        pltpu.SemaphoreType.DMA,
    ],
)


def _scat_body(src_hbm, dst_hbm, g_hbm, out_hbm, idx_s, idx_d, buf, acc_sh,
               gsem):
    bufs = [buf.at[pl.ds(b * CHUNK, CHUNK)] for b in range(NBUF)]
    gsems = [gsem.at[b] for b in range(NBUF)]
    c = lax.axis_index("c")
    s = lax.axis_index("s")
    w = c * 16 + s
    pltpu.sync_copy(src_hbm.at[w], idx_s)
    pltpu.sync_copy(dst_hbm.at[w], idx_d)
    _acc_slices(
        s,
        lambda o: pltpu.sync_copy(g_hbm.at[pl.ds(o, NSL)],
                                  acc_sh.at[pl.ds(o, NSL)]),
        lambda o: pltpu.sync_copy(g_hbm.at[pl.ds(o, NSL_LAST)],
                                  acc_sh.at[pl.ds(o, NSL_LAST)]),
    )
    plsc.subcore_barrier()

    def fire_gather(wi, b):
        pltpu.async_copy(g_hbm.at[idx_s.at[wi]], bufs[b], gsems[b])

    def wait_gather(wi, b):
        pltpu.make_async_copy(g_hbm.at[idx_s.at[wi]], bufs[b],
                              gsems[b]).wait()

    def fire_scatter(wi, b):
        pltpu.sync_copy(bufs[b], acc_sh.at[idx_d.at[wi]], add=True)

    def substep(wi, b, bp):
        wait_gather(wi, b)
        fire_scatter(wi, b)
        if bp is not None:
            fire_gather(wi + PF, bp)

    for wi in range(PF):
        fire_gather(wi, wi)
    for wi in range(PF):
        substep(wi, wi, wi + PF)

    def body(g, carry):
        w0 = g * NBUF + PF
        for k in range(NBUF):
            substep(w0 + k, (PF + k) % NBUF, k % NBUF)
        return carry

    lax.fori_loop(0, NGRP, body, 0)
    base = NGRP * NBUF + PF
    for k in range(TAIL):
        wi = base + k
        bp = (wi + PF) % NBUF if wi + PF < RPT else None
        substep(wi, wi % NBUF, bp)
    plsc.subcore_barrier()
    _acc_slices(
        s,
        lambda o: pltpu.sync_copy(acc_sh.at[pl.ds(o, NSL)],
                                  out_hbm.at[pl.ds(c * NP + o, NSL)]),
        lambda o: pltpu.sync_copy(acc_sh.at[pl.ds(o, NSL_LAST)],
                                  out_hbm.at[pl.ds(c * NP + o, NSL_LAST)]),
    )


def _make_scat(d):
    return pl.kernel(
        _scat_body,
        out_type=jax.ShapeDtypeStruct((2 * NP, d), jnp.float32),
        mesh=_mesh,
        compiler_params=_SC_PARAMS,
        scratch_types=[
            pltpu.VMEM((RPT, CHUNK), jnp.int32),
            pltpu.VMEM((RPT, CHUNK), jnp.int32),
            pltpu.VMEM((NBUF * CHUNK, d), jnp.float32),
            pltpu.VMEM_SHARED((NP, d), jnp.float32),
            pltpu.SemaphoreType.DMA((NBUF,)),
        ],
    )


_scat16 = _make_scat(D1)
_scat48 = _make_scat(D2P)


def _tc1_body(deg_ref, x_ref, w1_ref, g1_ref, dinv_ref):
    degp = deg_ref[...]
    deg = degp[:N, :] + degp[NP:NP + N, :] + 1.0
    dinv = lax.rsqrt(deg)
    h = jnp.dot(x_ref[...], w1_ref[...], preferred_element_type=jnp.float32)
    g1_ref[:N, :] = h * dinv
    g1_ref[N:, :] = jnp.zeros((NPAD, D1), jnp.float32)
    dinv_ref[...] = dinv


_tc1 = pl.pallas_call(
    _tc1_body,
    out_shape=(
        jax.ShapeDtypeStruct((NP, D1), jnp.float32),
        jax.ShapeDtypeStruct((N, 1), jnp.float32),
    ),
)


def _tc2_body(acc_ref, g1_ref, dinv_ref, w2_ref, b1_ref, g2_ref):
    acc = acc_ref[...]
    g1 = g1_ref[:N, :]
    dinv = dinv_ref[...]
    agg = acc[:N, :] + acc[NP:NP + N, :] - g1
    z = jnp.maximum(agg * dinv + b1_ref[...], 0.0)
    h2 = jnp.dot(z, w2_ref[...], preferred_element_type=jnp.float32)
    g2_ref[:N, :NC] = h2 * dinv
    g2_ref[:N, NC:] = jnp.zeros((N, D2P - NC), jnp.float32)
    g2_ref[N:, :] = jnp.zeros((NPAD, D2P), jnp.float32)


_tc2 = pl.pallas_call(
    _tc2_body,
    out_shape=jax.ShapeDtypeStruct((NP, D2P), jnp.float32),
)


def _tc3_body(acc_ref, g2_ref, dinv_ref, b2_ref, out_ref):
    acc = acc_ref[...]
    agg = acc[:N, :] + acc[NP:NP + N, :] - g2_ref[:N, :]
    o = agg[:, :NC] * dinv_ref[...] + b2_ref[...]
    m = jnp.max(o, axis=1, keepdims=True)
    e = o - m
    lse = jnp.log(jnp.sum(jnp.exp(e), axis=1, keepdims=True))
    out_ref[...] = e - lse


_tc3 = pl.pallas_call(
    _tc3_body,
    out_shape=jax.ShapeDtypeStruct((N, NC), jnp.float32),
)


def kernel(x, edge_index, W1, b1, W2, b2):
    pad = jnp.tile(N + jnp.arange(NPAD, dtype=jnp.int32), (EP - E) // NPAD)
    src3d = jnp.concatenate([edge_index[0], pad]).reshape(32, RPT, CHUNK)
    dst3d = jnp.concatenate([edge_index[1], pad]).reshape(32, RPT, CHUNK)
    aux = jnp.concatenate(
        [jnp.zeros((NP, 1), jnp.float32), jnp.ones((CHUNK, 1), jnp.float32)])
    degp = _deg_call(dst3d, aux)
    g1, dinv = _tc1(degp, x, W1)
    acc1 = _scat16(src3d, dst3d, g1)
    g2 = _tc2(acc1, g1, dinv, W2, b1.reshape(1, D1))
    acc2 = _scat48(src3d, dst3d, g2)
    return _tc3(acc2, g2, dinv, b2.reshape(1, NC))


# speedup vs baseline: 1.1489x; 1.0389x over previous
import jax
import jax.numpy as jnp
from jax import lax
from jax.experimental import pallas as pl
from jax.experimental.pallas import tpu as pltpu
from jax.experimental.pallas import tpu_sc as plsc

N = 10000
E = 320000
D1 = 16
D2P = 48
NC = 40
NPAD = 0
NP = N + NPAD
CHUNK = 80
RPT = 125
EP = 32 * RPT * CHUNK
NSL = 640
NSL_LAST = NP - 15 * NSL

PF = 5
NBUF = 10
NGRP = (RPT - 2 * PF) // NBUF
TAIL = RPT - PF - NGRP * NBUF

_mesh = plsc.VectorSubcoreMesh(core_axis_name="c", subcore_axis_name="s")
_SC_PARAMS = pltpu.CompilerParams(use_tc_tiling_on_sc=False)


def _acc_slices(s, copy_main, copy_last):
    @pl.when(s < 15)
    def _():
        copy_main(s * NSL)

    @pl.when(s == 15)
    def _():
        copy_last(15 * NSL)


def _deg_body(dst_hbm, aux_hbm, out_hbm, idx_v, ones_v, acc_sh, sem):
    c = lax.axis_index("c")
    s = lax.axis_index("s")
    w = c * 16 + s
    pltpu.sync_copy(dst_hbm.at[w], idx_v)
    pltpu.sync_copy(aux_hbm.at[pl.ds(NP, CHUNK)], ones_v)
    _acc_slices(
        s,
        lambda o: pltpu.sync_copy(aux_hbm.at[pl.ds(o, NSL)],
                                  acc_sh.at[pl.ds(o, NSL)]),
        lambda o: pltpu.sync_copy(aux_hbm.at[pl.ds(o, NSL_LAST)],
                                  acc_sh.at[pl.ds(o, NSL_LAST)]),
    )
    plsc.subcore_barrier()

    def body(i, carry):
        pltpu.sync_copy(ones_v, acc_sh.at[idx_v.at[i]], add=True)
        return carry

    lax.fori_loop(0, RPT, body, 0)
    plsc.subcore_barrier()
    _acc_slices(
        s,
        lambda o: pltpu.sync_copy(acc_sh.at[pl.ds(o, NSL)],
                                  out_hbm.at[pl.ds(c * NP + o, NSL)]),
        lambda o: pltpu.sync_copy(acc_sh.at[pl.ds(o, NSL_LAST)],
                                  out_hbm.at[pl.ds(c * NP + o, NSL_LAST)]),
    )


_deg_call = pl.kernel(
    _deg_body,
    out_type=jax.ShapeDtypeStruct((2 * NP, 1), jnp.float32),
    mesh=_mesh,
    compiler_params=_SC_PARAMS,
    scratch_types=[
        pltpu.VMEM((RPT, CHUNK), jnp.int32),
        pltpu.VMEM((CHUNK, 1), jnp.float32),
        pltpu.VMEM_SHARED((NP, 1), jnp.float32),
        pltpu.SemaphoreType.DMA,
    ],
)


def _scat_body(src_hbm, dst_hbm, g_hbm, out_hbm, idx_s, idx_d, buf, acc_sh,
               gsem):
    bufs = [buf.at[pl.ds(b * CHUNK, CHUNK)] for b in range(NBUF)]
    gsems = [gsem.at[b] for b in range(NBUF)]
    c = lax.axis_index("c")
    s = lax.axis_index("s")
    w = c * 16 + s
    pltpu.sync_copy(src_hbm.at[w], idx_s)
    pltpu.sync_copy(dst_hbm.at[w], idx_d)
    _acc_slices(
        s,
        lambda o: pltpu.sync_copy(g_hbm.at[pl.ds(o, NSL)],
                                  acc_sh.at[pl.ds(o, NSL)]),
        lambda o: pltpu.sync_copy(g_hbm.at[pl.ds(o, NSL_LAST)],
                                  acc_sh.at[pl.ds(o, NSL_LAST)]),
    )
    plsc.subcore_barrier()

    def fire_gather(wi, b):
        pltpu.async_copy(g_hbm.at[idx_s.at[wi]], bufs[b], gsems[b])

    def wait_gather(wi, b):
        pltpu.make_async_copy(g_hbm.at[idx_s.at[wi]], bufs[b],
                              gsems[b]).wait()

    def fire_scatter(wi, b):
        pltpu.sync_copy(bufs[b], acc_sh.at[idx_d.at[wi]], add=True)

    def substep(wi, b, bp):
        wait_gather(wi, b)
        fire_scatter(wi, b)
        if bp is not None:
            fire_gather(wi + PF, bp)

    for wi in range(PF):
        fire_gather(wi, wi)
    for wi in range(PF):
        substep(wi, wi, wi + PF)

    def body(g, carry):
        w0 = g * NBUF + PF
        for k in range(NBUF):
            substep(w0 + k, (PF + k) % NBUF, k % NBUF)
        return carry

    lax.fori_loop(0, NGRP, body, 0)
    base = NGRP * NBUF + PF
    for k in range(TAIL):
        wi = base + k
        bp = (wi + PF) % NBUF if wi + PF < RPT else None
        substep(wi, wi % NBUF, bp)
    plsc.subcore_barrier()
    _acc_slices(
        s,
        lambda o: pltpu.sync_copy(acc_sh.at[pl.ds(o, NSL)],
                                  out_hbm.at[pl.ds(c * NP + o, NSL)]),
        lambda o: pltpu.sync_copy(acc_sh.at[pl.ds(o, NSL_LAST)],
                                  out_hbm.at[pl.ds(c * NP + o, NSL_LAST)]),
    )


def _make_scat(d):
    return pl.kernel(
        _scat_body,
        out_type=jax.ShapeDtypeStruct((2 * NP, d), jnp.float32),
        mesh=_mesh,
        compiler_params=_SC_PARAMS,
        scratch_types=[
            pltpu.VMEM((RPT, CHUNK), jnp.int32),
            pltpu.VMEM((RPT, CHUNK), jnp.int32),
            pltpu.VMEM((NBUF * CHUNK, d), jnp.float32),
            pltpu.VMEM_SHARED((NP, d), jnp.float32),
            pltpu.SemaphoreType.DMA((NBUF,)),
        ],
    )


_scat16 = _make_scat(D1)
_scat48 = _make_scat(D2P)


def _tc1_body(deg_ref, x_ref, w1_ref, g1_ref, dinv_ref):
    degp = deg_ref[...]
    deg = degp[:N, :] + degp[NP:NP + N, :] + 1.0
    dinv = lax.rsqrt(deg)
    h = jnp.dot(x_ref[...], w1_ref[...], preferred_element_type=jnp.float32)
    g1_ref[:N, :] = h * dinv
    if NPAD:
        g1_ref[N:, :] = jnp.zeros((NPAD, D1), jnp.float32)
    dinv_ref[...] = dinv


_tc1 = pl.pallas_call(
    _tc1_body,
    out_shape=(
        jax.ShapeDtypeStruct((NP, D1), jnp.float32),
        jax.ShapeDtypeStruct((N, 1), jnp.float32),
    ),
)


def _tc2_body(acc_ref, g1_ref, dinv_ref, w2_ref, b1_ref, g2_ref):
    acc = acc_ref[...]
    g1 = g1_ref[:N, :]
    dinv = dinv_ref[...]
    agg = acc[:N, :] + acc[NP:NP + N, :] - g1
    z = jnp.maximum(agg * dinv + b1_ref[...], 0.0)
    h2 = jnp.dot(z, w2_ref[...], preferred_element_type=jnp.float32)
    g2_ref[:N, :NC] = h2 * dinv
    g2_ref[:N, NC:] = jnp.zeros((N, D2P - NC), jnp.float32)
    if NPAD:
        g2_ref[N:, :] = jnp.zeros((NPAD, D2P), jnp.float32)


_tc2 = pl.pallas_call(
    _tc2_body,
    out_shape=jax.ShapeDtypeStruct((NP, D2P), jnp.float32),
)


def _tc3_body(acc_ref, g2_ref, dinv_ref, b2_ref, out_ref):
    acc = acc_ref[...]
    agg = acc[:N, :] + acc[NP:NP + N, :] - g2_ref[:N, :]
    o = agg[:, :NC] * dinv_ref[...] + b2_ref[...]
    m = jnp.max(o, axis=1, keepdims=True)
    e = o - m
    lse = jnp.log(jnp.sum(jnp.exp(e), axis=1, keepdims=True))
    out_ref[...] = e - lse


_tc3 = pl.pallas_call(
    _tc3_body,
    out_shape=jax.ShapeDtypeStruct((N, NC), jnp.float32),
)


def kernel(x, edge_index, W1, b1, W2, b2):
    if EP > E:
        pad = jnp.tile(N + jnp.arange(NPAD, dtype=jnp.int32),
                       (EP - E) // NPAD)
        src3d = jnp.concatenate([edge_index[0], pad]).reshape(32, RPT, CHUNK)
        dst3d = jnp.concatenate([edge_index[1], pad]).reshape(32, RPT, CHUNK)
    else:
        src3d = edge_index[0].reshape(32, RPT, CHUNK)
        dst3d = edge_index[1].reshape(32, RPT, CHUNK)
    aux = jnp.concatenate(
        [jnp.zeros((NP, 1), jnp.float32), jnp.ones((CHUNK, 1), jnp.float32)])
    degp = _deg_call(dst3d, aux)
    g1, dinv = _tc1(degp, x, W1)
    acc1 = _scat16(src3d, dst3d, g1)
    g2 = _tc2(acc1, g1, dinv, W2, b1.reshape(1, D1))
    acc2 = _scat48(src3d, dst3d, g2)
    return _tc3(acc2, g2, dinv, b2.reshape(1, NC))


# speedup vs baseline: 1.2009x; 1.0452x over previous
import jax
import jax.numpy as jnp
from jax import lax
from jax.experimental import pallas as pl
from jax.experimental.pallas import tpu as pltpu
from jax.experimental.pallas import tpu_sc as plsc

N = 10000
E = 320000
D1 = 16
D2P = 48
NC = 40
NPAD = 0
NP = N + NPAD
CHUNK = 80
RPT = 125
EP = 32 * RPT * CHUNK
NSL = 640
NSL_LAST = NP - 15 * NSL

PF = 8
NBUF = 10
NGRP = (RPT - 2 * PF) // NBUF
TAIL = RPT - PF - NGRP * NBUF

_mesh = plsc.VectorSubcoreMesh(core_axis_name="c", subcore_axis_name="s")
_SC_PARAMS = pltpu.CompilerParams(use_tc_tiling_on_sc=False)


def _acc_slices(s, copy_main, copy_last):
    @pl.when(s < 15)
    def _():
        copy_main(s * NSL)

    @pl.when(s == 15)
    def _():
        copy_last(15 * NSL)


def _deg_body(dst_hbm, aux_hbm, out_hbm, idx_v, ones_v, acc_sh, sem):
    c = lax.axis_index("c")
    s = lax.axis_index("s")
    w = c * 16 + s
    pltpu.sync_copy(dst_hbm.at[w], idx_v)
    pltpu.sync_copy(aux_hbm.at[pl.ds(NP, CHUNK)], ones_v)
    _acc_slices(
        s,
        lambda o: pltpu.sync_copy(aux_hbm.at[pl.ds(o, NSL)],
                                  acc_sh.at[pl.ds(o, NSL)]),
        lambda o: pltpu.sync_copy(aux_hbm.at[pl.ds(o, NSL_LAST)],
                                  acc_sh.at[pl.ds(o, NSL_LAST)]),
    )
    plsc.subcore_barrier()

    def body(i, carry):
        pltpu.sync_copy(ones_v, acc_sh.at[idx_v.at[i]], add=True)
        return carry

    lax.fori_loop(0, RPT, body, 0)
    plsc.subcore_barrier()
    _acc_slices(
        s,
        lambda o: pltpu.sync_copy(acc_sh.at[pl.ds(o, NSL)],
                                  out_hbm.at[pl.ds(c * NP + o, NSL)]),
        lambda o: pltpu.sync_copy(acc_sh.at[pl.ds(o, NSL_LAST)],
                                  out_hbm.at[pl.ds(c * NP + o, NSL_LAST)]),
    )


_deg_call = pl.kernel(
    _deg_body,
    out_type=jax.ShapeDtypeStruct((2 * NP, 1), jnp.float32),
    mesh=_mesh,
    compiler_params=_SC_PARAMS,
    scratch_types=[
        pltpu.VMEM((RPT, CHUNK), jnp.int32),
        pltpu.VMEM((CHUNK, 1), jnp.float32),
        pltpu.VMEM_SHARED((NP, 1), jnp.float32),
        pltpu.SemaphoreType.DMA,
    ],
)


def _scat_body(src_hbm, dst_hbm, g_hbm, out_hbm, idx_s, idx_d, buf, acc_sh,
               gsem):
    bufs = [buf.at[pl.ds(b * CHUNK, CHUNK)] for b in range(NBUF)]
    gsems = [gsem.at[b] for b in range(NBUF)]
    c = lax.axis_index("c")
    s = lax.axis_index("s")
    w = c * 16 + s
    pltpu.sync_copy(src_hbm.at[w], idx_s)
    pltpu.sync_copy(dst_hbm.at[w], idx_d)
    _acc_slices(
        s,
        lambda o: pltpu.sync_copy(g_hbm.at[pl.ds(o, NSL)],
                                  acc_sh.at[pl.ds(o, NSL)]),
        lambda o: pltpu.sync_copy(g_hbm.at[pl.ds(o, NSL_LAST)],
                                  acc_sh.at[pl.ds(o, NSL_LAST)]),
    )
    plsc.subcore_barrier()

    def fire_gather(wi, b):
        pltpu.async_copy(g_hbm.at[idx_s.at[wi]], bufs[b], gsems[b])

    def wait_gather(wi, b):
        pltpu.make_async_copy(g_hbm.at[idx_s.at[wi]], bufs[b],
                              gsems[b]).wait()

    def fire_scatter(wi, b):
        pltpu.sync_copy(bufs[b], acc_sh.at[idx_d.at[wi]], add=True)

    def substep(wi, b, bp):
        wait_gather(wi, b)
        fire_scatter(wi, b)
        if bp is not None:
            fire_gather(wi + PF, bp)

    for wi in range(PF):
        fire_gather(wi, wi % NBUF)
    for wi in range(PF):
        substep(wi, wi % NBUF, (wi + PF) % NBUF)

    def body(g, carry):
        w0 = g * NBUF + PF
        for k in range(NBUF):
            substep(w0 + k, (PF + k) % NBUF, (2 * PF + k) % NBUF)
        return carry

    lax.fori_loop(0, NGRP, body, 0)
    base = NGRP * NBUF + PF
    for k in range(TAIL):
        wi = base + k
        bp = (wi + PF) % NBUF if wi + PF < RPT else None
        substep(wi, wi % NBUF, bp)
    plsc.subcore_barrier()
    _acc_slices(
        s,
        lambda o: pltpu.sync_copy(acc_sh.at[pl.ds(o, NSL)],
                                  out_hbm.at[pl.ds(c * NP + o, NSL)]),
        lambda o: pltpu.sync_copy(acc_sh.at[pl.ds(o, NSL_LAST)],
                                  out_hbm.at[pl.ds(c * NP + o, NSL_LAST)]),
    )


def _make_scat(d):
    return pl.kernel(
        _scat_body,
        out_type=jax.ShapeDtypeStruct((2 * NP, d), jnp.float32),
        mesh=_mesh,
        compiler_params=_SC_PARAMS,
        scratch_types=[
            pltpu.VMEM((RPT, CHUNK), jnp.int32),
            pltpu.VMEM((RPT, CHUNK), jnp.int32),
            pltpu.VMEM((NBUF * CHUNK, d), jnp.float32),
            pltpu.VMEM_SHARED((NP, d), jnp.float32),
            pltpu.SemaphoreType.DMA((NBUF,)),
        ],
    )


_scat16 = _make_scat(D1)
_scat48 = _make_scat(D2P)


def _tc1_body(deg_ref, x_ref, w1_ref, g1_ref, dinv_ref):
    degp = deg_ref[...]
    deg = degp[:N, :] + degp[NP:NP + N, :] + 1.0
    dinv = lax.rsqrt(deg)
    h = jnp.dot(x_ref[...], w1_ref[...], preferred_element_type=jnp.float32)
    g1_ref[:N, :] = h * dinv
    if NPAD:
        g1_ref[N:, :] = jnp.zeros((NPAD, D1), jnp.float32)
    dinv_ref[...] = dinv


_tc1 = pl.pallas_call(
    _tc1_body,
    out_shape=(
        jax.ShapeDtypeStruct((NP, D1), jnp.float32),
        jax.ShapeDtypeStruct((N, 1), jnp.float32),
    ),
)


def _tc2_body(acc_ref, g1_ref, dinv_ref, w2_ref, b1_ref, g2_ref):
    acc = acc_ref[...]
    g1 = g1_ref[:N, :]
    dinv = dinv_ref[...]
    agg = acc[:N, :] + acc[NP:NP + N, :] - g1
    z = jnp.maximum(agg * dinv + b1_ref[...], 0.0)
    h2 = jnp.dot(z, w2_ref[...], preferred_element_type=jnp.float32)
    g2_ref[:N, :NC] = h2 * dinv
    g2_ref[:N, NC:] = jnp.zeros((N, D2P - NC), jnp.float32)
    if NPAD:
        g2_ref[N:, :] = jnp.zeros((NPAD, D2P), jnp.float32)


_tc2 = pl.pallas_call(
    _tc2_body,
    out_shape=jax.ShapeDtypeStruct((NP, D2P), jnp.float32),
)


def _tc3_body(acc_ref, g2_ref, dinv_ref, b2_ref, out_ref):
    acc = acc_ref[...]
    agg = acc[:N, :] + acc[NP:NP + N, :] - g2_ref[:N, :]
    o = agg[:, :NC] * dinv_ref[...] + b2_ref[...]
    m = jnp.max(o, axis=1, keepdims=True)
    e = o - m
    lse = jnp.log(jnp.sum(jnp.exp(e), axis=1, keepdims=True))
    out_ref[...] = e - lse


_tc3 = pl.pallas_call(
    _tc3_body,
    out_shape=jax.ShapeDtypeStruct((N, NC), jnp.float32),
)


def kernel(x, edge_index, W1, b1, W2, b2):
    if EP > E:
        pad = jnp.tile(N + jnp.arange(NPAD, dtype=jnp.int32),
                       (EP - E) // NPAD)
        src3d = jnp.concatenate([edge_index[0], pad]).reshape(32, RPT, CHUNK)
        dst3d = jnp.concatenate([edge_index[1], pad]).reshape(32, RPT, CHUNK)
    else:
        src3d = edge_index[0].reshape(32, RPT, CHUNK)
        dst3d = edge_index[1].reshape(32, RPT, CHUNK)
    aux = jnp.concatenate(
        [jnp.zeros((NP, 1), jnp.float32), jnp.ones((CHUNK, 1), jnp.float32)])
    degp = _deg_call(dst3d, aux)
    g1, dinv = _tc1(degp, x, W1)
    acc1 = _scat16(src3d, dst3d, g1)
    g2 = _tc2(acc1, g1, dinv, W2, b1.reshape(1, D1))
    acc2 = _scat48(src3d, dst3d, g2)
    return _tc3(acc2, g2, dinv, b2.reshape(1, NC))
